# baseline (speedup 1.0000x reference)
import functools

import jax
import jax.numpy as jnp
from jax.experimental import pallas as pl
from jax.experimental.pallas import tpu as pltpu

_EPS = 1e-12
_ATOL = 1e-8
_RTOL = 1e-5
_CHUNK = 128
_RSTRIP = 128


def _row_ss(x):
    rows, d = x.shape
    rs = min(_RSTRIP, rows)
    outs = []
    for r in range(0, rows, rs):
        acc = None
        for c in range(0, d, _CHUNK):
            blk = x[r:r + rs, c:c + _CHUNK]
            sq = blk * blk
            acc = sq if acc is None else acc + sq
        outs.append(jnp.sum(acc, axis=1, keepdims=True))
    return outs[0] if len(outs) == 1 else jnp.concatenate(outs, axis=0)


def _normalize_rows(x):
    return x / jnp.maximum(jnp.sqrt(_row_ss(x)), _EPS)


def _row_ss_ref(ref):
    rows, d = ref.shape

    def body(c, acc):
        blk = ref[:, pl.ds(c * _CHUNK, _CHUNK)]
        return acc + blk * blk

    acc = jax.lax.fori_loop(0, d // _CHUNK, body,
                            jnp.zeros((rows, _CHUNK), jnp.float32))
    return jnp.sum(acc, axis=1, keepdims=True)


def _rank_kernel(q_ref, t_ref, d_ref, out_ref, qn_ref, te_ref):
    k = pl.program_id(0)

    @pl.when(k == 0)
    def _init():
        qn = _normalize_rows(q_ref[...])
        tn = _normalize_rows(t_ref[...])
        qn_ref[...] = qn
        m = jax.lax.dot_general(qn, tn, (((1,), (1,)), ((), ())))
        nq = m.shape[0]
        eye = (jax.lax.broadcasted_iota(jnp.int32, (nq, nq), 0)
               == jax.lax.broadcasted_iota(jnp.int32, (nq, nq), 1))
        thr = jnp.sum(jnp.where(eye, m, 0.0), axis=0, keepdims=True)
        te_ref[...] = thr - (_ATOL + _RTOL * jnp.abs(thr))

    s = jax.lax.dot_general(d_ref[...], qn_ref[...],
                            (((1,), (1,)), ((), ())))
    norm = jnp.maximum(jnp.sqrt(_row_ss_ref(d_ref)), _EPS)
    ge = s >= te_ref[...] * norm
    cnt = jnp.sum(ge.astype(jnp.int32), axis=0, keepdims=True)

    @pl.when(k == 0)
    def _first():
        out_ref[...] = cnt - 1

    @pl.when(k != 0)
    def _rest():
        out_ref[...] = out_ref[...] + cnt


@functools.partial(jax.jit, static_argnames=("block",))
def _rank(queries, truths, data, block=512):
    n, d = data.shape
    nq = queries.shape[0]
    return pl.pallas_call(
        _rank_kernel,
        grid=(n // block,),
        in_specs=[
            pl.BlockSpec((nq, d), lambda k: (0, 0)),
            pl.BlockSpec((nq, d), lambda k: (0, 0)),
            pl.BlockSpec((block, d), lambda k: (k, 0)),
        ],
        out_specs=pl.BlockSpec((1, nq), lambda k: (0, 0)),
        out_shape=jax.ShapeDtypeStruct((1, nq), jnp.int32),
        scratch_shapes=[
            pltpu.VMEM((nq, d), jnp.float32),
            pltpu.VMEM((1, nq), jnp.float32),
        ],
        compiler_params=pltpu.CompilerParams(
            dimension_semantics=("arbitrary",),
        ),
    )(queries, truths, data)


def kernel(queries, truths, data):
    return _rank(queries, truths, data)


# speedup vs baseline: 1.1457x; 1.1457x over previous
import functools

import jax
import jax.numpy as jnp
from jax.experimental import pallas as pl
from jax.experimental.pallas import tpu as pltpu

_EPS = 1e-12
_ATOL = 1e-8
_RTOL = 1e-5
_CHUNK = 128
_UNROLL = 4
_RSTRIP = 128


def _row_ss(x):
    rows, d = x.shape
    rs = min(_RSTRIP, rows)
    outs = []
    for r in range(0, rows, rs):
        acc = None
        for c in range(0, d, _CHUNK):
            blk = x[r:r + rs, c:c + _CHUNK]
            sq = blk * blk
            acc = sq if acc is None else acc + sq
        outs.append(jnp.sum(acc, axis=1, keepdims=True))
    return outs[0] if len(outs) == 1 else jnp.concatenate(outs, axis=0)


def _normalize_rows(x):
    return x / jnp.maximum(jnp.sqrt(_row_ss(x)), _EPS)


def _row_ss_ref(ref):
    rows, d = ref.shape

    def body(c, acc):
        blk = ref[:, pl.ds(c * _CHUNK, _CHUNK)]
        return acc + blk * blk

    acc = jax.lax.fori_loop(0, d // _CHUNK, body,
                            jnp.zeros((rows, _CHUNK), jnp.float32),
                            unroll=_UNROLL)
    return jnp.sum(acc, axis=1, keepdims=True)


def _rank_kernel(q_ref, t_ref, d_ref, out_ref, qn_ref, te_ref):
    k = pl.program_id(0)

    @pl.when(k == 0)
    def _init():
        qn = _normalize_rows(q_ref[...])
        tn = _normalize_rows(t_ref[...])
        qn_ref[...] = qn
        m = jax.lax.dot_general(qn, tn, (((1,), (1,)), ((), ())))
        nq = m.shape[0]
        eye = (jax.lax.broadcasted_iota(jnp.int32, (nq, nq), 0)
               == jax.lax.broadcasted_iota(jnp.int32, (nq, nq), 1))
        thr = jnp.sum(jnp.where(eye, m, 0.0), axis=0, keepdims=True)
        te_ref[...] = thr - (_ATOL + _RTOL * jnp.abs(thr))

    s = jax.lax.dot_general(d_ref[...], qn_ref[...],
                            (((1,), (1,)), ((), ())))
    norm = jnp.maximum(jnp.sqrt(_row_ss_ref(d_ref)), _EPS)
    ge = s >= te_ref[...] * norm
    cnt = jnp.sum(ge.astype(jnp.int32), axis=0, keepdims=True)

    @pl.when(k == 0)
    def _first():
        out_ref[...] = cnt - 1

    @pl.when(k != 0)
    def _rest():
        out_ref[...] = out_ref[...] + cnt


@functools.partial(jax.jit, static_argnames=("block",))
def _rank(queries, truths, data, block=512):
    n, d = data.shape
    nq = queries.shape[0]
    return pl.pallas_call(
        _rank_kernel,
        grid=(n // block,),
        in_specs=[
            pl.BlockSpec((nq, d), lambda k: (0, 0)),
            pl.BlockSpec((nq, d), lambda k: (0, 0)),
            pl.BlockSpec((block, d), lambda k: (k, 0)),
        ],
        out_specs=pl.BlockSpec((1, nq), lambda k: (0, 0)),
        out_shape=jax.ShapeDtypeStruct((1, nq), jnp.int32),
        scratch_shapes=[
            pltpu.VMEM((nq, d), jnp.float32),
            pltpu.VMEM((1, nq), jnp.float32),
        ],
        compiler_params=pltpu.CompilerParams(
            dimension_semantics=("arbitrary",),
        ),
    )(queries, truths, data)


def kernel(queries, truths, data):
    return _rank(queries, truths, data)


# speedup vs baseline: 1.5127x; 1.3203x over previous
import functools

import jax
import jax.numpy as jnp
from jax.experimental import pallas as pl
from jax.experimental.pallas import tpu as pltpu

_EPS = 1e-12
_ATOL = 1e-8
_RTOL = 1e-5
_KCHUNK = 512


def _row_ss(x):
    return jnp.sum(x * x, axis=1, keepdims=True)


def _normalize_rows(x):
    return x / jnp.maximum(jnp.sqrt(_row_ss(x)), _EPS)


def _rank_kernel(q_ref, t_ref, d_ref, out_ref, qn_ref, te_ref, cnt_ref):
    k = pl.program_id(0)

    @pl.when(k == 0)
    def _init():
        qn = _normalize_rows(q_ref[...])
        tn = _normalize_rows(t_ref[...])
        qn_ref[...] = qn
        m = jax.lax.dot_general(qn, tn, (((1,), (1,)), ((), ())))
        nq = m.shape[0]
        eye = (jax.lax.broadcasted_iota(jnp.int32, (nq, nq), 0)
               == jax.lax.broadcasted_iota(jnp.int32, (nq, nq), 1))
        thr = jnp.sum(jnp.where(eye, m, 0.0), axis=0, keepdims=True)
        te_ref[...] = thr - (_ATOL + _RTOL * jnp.abs(thr))

    dim = d_ref.shape[1]
    s = None
    ss = None
    for c in range(0, dim, _KCHUNK):
        dc = d_ref[:, c:c + _KCHUNK]
        qc = qn_ref[:, c:c + _KCHUNK]
        ps = jax.lax.dot_general(dc, qc, (((1,), (1,)), ((), ())))
        pss = _row_ss(dc)
        s = ps if s is None else s + ps
        ss = pss if ss is None else ss + pss
    norm = jnp.maximum(jnp.sqrt(ss), _EPS)
    ge = s >= te_ref[...] * norm
    cnt = jnp.sum(ge.astype(jnp.int32), axis=0, keepdims=True)

    @pl.when(k == 0)
    def _first():
        cnt_ref[...] = cnt - 1

    @pl.when(k != 0)
    def _rest():
        cnt_ref[...] = cnt_ref[...] + cnt

    @pl.when(k == pl.num_programs(0) - 1)
    def _emit():
        out_ref[...] = cnt_ref[...]


@functools.partial(jax.jit, static_argnames=("block",))
def _rank(queries, truths, data, block=512):
    n, d = data.shape
    nq = queries.shape[0]
    return pl.pallas_call(
        _rank_kernel,
        grid=(n // block,),
        in_specs=[
            pl.BlockSpec((nq, d), lambda k: (0, 0)),
            pl.BlockSpec((nq, d), lambda k: (0, 0)),
            pl.BlockSpec((block, d), lambda k: (k, 0)),
        ],
        out_specs=pl.BlockSpec((1, nq), lambda k: (0, 0)),
        out_shape=jax.ShapeDtypeStruct((1, nq), jnp.int32),
        scratch_shapes=[
            pltpu.VMEM((nq, d), jnp.float32),
            pltpu.VMEM((1, nq), jnp.float32),
            pltpu.VMEM((1, nq), jnp.int32),
        ],
        compiler_params=pltpu.CompilerParams(
            dimension_semantics=("arbitrary",),
        ),
    )(queries, truths, data)


def kernel(queries, truths, data):
    return _rank(queries, truths, data)
